# baseline (speedup 1.0000x reference)
import functools

import jax
import jax.numpy as jnp
from jax import lax
from jax.experimental import pallas as pl
from jax.experimental.pallas import tpu as pltpu
from jax.experimental.pallas import tpu_sc as plsc

BATCH = 4096
SEQ = 200
EMBED = 64
NC = 2
NS = 16
NW = NC * NS
BLK = BATCH // NW
SHI = SEQ // 8
NQ = SEQ * NW // 2

H = 4
NGROUP = SEQ // H
NPAIR = NGROUP // 2


def _sc_body(ids_hbm, table_hbm, inter_hbm, ibuf, gbuf, *sems, seq):
    ngroup = seq // H
    npair = ngroup // 2
    gsem = sems[:2 * H]
    wsem = sems[2 * H:]
    w = lax.axis_index("s") * NC + lax.axis_index("c")
    q0 = w // 2
    colp = w % 2

    pltpu.sync_copy(ids_hbm.at[:, w], ibuf)

    def gather_start(s, b):
        pltpu.async_copy(table_hbm.at[ibuf.at[s // 8, s % 8]],
                         gbuf.at[b], gsem[b])

    def gather_wait(s, b):
        pltpu.make_async_copy(table_hbm.at[ibuf.at[s // 8, s % 8]],
                              gbuf.at[b], gsem[b]).wait()

    def _dst(s):
        return inter_hbm.at[s * 16 + q0, :, pl.ds(64 * colp, 64)]

    def write_start(s, b):
        pltpu.async_copy(gbuf.at[b], _dst(s), wsem[b])

    def write_wait(s, b):
        pltpu.make_async_copy(gbuf.at[b], _dst(s), wsem[b]).wait()

    for b in range(H):
        gather_start(b, b)

    def body(u, carry):
        for p in (0, 1):
            g = 2 * u + p
            c0 = g * H
            for b in range(H):
                gather_wait(c0 + b, p * H + b)
            for b in range(H):
                write_start(c0 + b, p * H + b)

            @pl.when(g >= 1)
            def _(c0=c0, p=p):
                for b in range(H):
                    write_wait(c0 - H + b, (1 - p) * H + b)

            @pl.when(g < ngroup - 1)
            def _(c0=c0, p=p):
                for b in range(H):
                    gather_start(c0 + H + b, (1 - p) * H + b)
        return carry

    lax.fori_loop(0, npair, body, 0)
    for b in range(H):
        write_wait(seq - H + b, H + b)


QB = 16


def _tc_body(in_ref, out_ref):
    for i in range(QB):
        t = in_ref[i].T
        t4 = t.reshape(2, 8, 8, BLK)
        out_ref[0, :, 2 * i, :, :] = t4[0]
        out_ref[0, :, 2 * i + 1, :, :] = t4[1]


def _sc_call(ids_slab, tlin, nsh):
    seq = 8 * nsh
    mesh = plsc.VectorSubcoreMesh(core_axis_name="c", subcore_axis_name="s")
    return pl.kernel(
        functools.partial(_sc_body, seq=seq),
        out_type=jax.ShapeDtypeStruct((seq * 16, BLK, BLK), jnp.float32),
        mesh=mesh,
        scratch_types=(
            [pltpu.VMEM((nsh, 8, BLK), jnp.int32),
             pltpu.VMEM((2 * H, BLK, EMBED), jnp.float32)]
            + [pltpu.SemaphoreType.DMA] * (4 * H)
        ),
        compiler_params=pltpu.CompilerParams(use_tc_tiling_on_sc=False,
                                             needs_layout_passes=False),
    )(ids_slab, tlin)


def _tc_call(inter, seq):
    return pl.pallas_call(
        _tc_body,
        out_shape=jax.ShapeDtypeStruct((seq, 8, NW, 8, BLK), jnp.float32),
        grid=(seq,),
        in_specs=[pl.BlockSpec((QB, BLK, BLK), lambda t: (t, 0, 0))],
        out_specs=pl.BlockSpec((1, 8, 2 * QB, 8, BLK),
                               lambda t: (t, 0, 0, 0, 0)),
    )(inter)


SPLIT = 12


def kernel(input_ids, table):
    ids4 = input_ids.T.reshape(SHI, 8, NW, BLK).transpose(0, 2, 1, 3)
    t2 = jax.lax.optimization_barrier(table.reshape(500000, 2 * EMBED))
    tlin = t2.reshape(1000000, EMBED)

    ia = _sc_call(ids4[:SPLIT], tlin, SPLIT)
    ib = _sc_call(ids4[SPLIT:], tlin, SHI - SPLIT)
    oa = _tc_call(ia, 8 * SPLIT)
    ob = _tc_call(ib, 8 * (SHI - SPLIT))
    out5 = jnp.concatenate([oa, ob], axis=0)

    return out5.transpose(2, 4, 0, 1, 3).reshape(BATCH, SEQ, EMBED)


# speedup vs baseline: 1.0970x; 1.0970x over previous
import jax
import jax.numpy as jnp
from jax import lax
from jax.experimental import pallas as pl
from jax.experimental.pallas import tpu as pltpu
from jax.experimental.pallas import tpu_sc as plsc

BATCH = 4096
SEQ = 200
EMBED = 64
NC = 2
NS = 16
NW = NC * NS
BLK = BATCH // NW
SHI = SEQ // 8
NQ = SEQ * NW // 2

H = 4
NGROUP = SEQ // H
NPAIR = NGROUP // 2


def _sc_body(ids_hbm, table_hbm, inter_hbm, ibuf, gbuf, *sems):
    gsem = sems[:2 * H]
    wsem = sems[2 * H:]
    w = lax.axis_index("s") * NC + lax.axis_index("c")
    q0 = w // 2
    colp = w % 2

    pltpu.sync_copy(ids_hbm.at[:, w], ibuf)

    def gather_start(s, b):
        pltpu.async_copy(table_hbm.at[ibuf.at[s // 8, s % 8]],
                         gbuf.at[b], gsem[b])

    def gather_wait(s, b):
        pltpu.make_async_copy(table_hbm.at[ibuf.at[s // 8, s % 8]],
                              gbuf.at[b], gsem[b]).wait()

    def _dst(s):
        return inter_hbm.at[s * 16 + q0, :, pl.ds(64 * colp, 64)]

    def write_start(s, b):
        pltpu.async_copy(gbuf.at[b], _dst(s), wsem[b])

    def write_wait(s, b):
        pltpu.make_async_copy(gbuf.at[b], _dst(s), wsem[b]).wait()

    for b in range(H):
        gather_start(b, b)

    def body(u, carry):
        for p in (0, 1):
            g = 2 * u + p
            c0 = g * H
            for b in range(H):
                gather_wait(c0 + b, p * H + b)
            for b in range(H):
                write_start(c0 + b, p * H + b)

            @pl.when(g >= 1)
            def _(c0=c0, p=p):
                for b in range(H):
                    write_wait(c0 - H + b, (1 - p) * H + b)

            @pl.when(g < NGROUP - 1)
            def _(c0=c0, p=p):
                for b in range(H):
                    gather_start(c0 + H + b, (1 - p) * H + b)
        return carry

    lax.fori_loop(0, NPAIR, body, 0)
    for b in range(H):
        write_wait(SEQ - H + b, H + b)


QB = 16


def _tc_body(in_ref, out_ref):
    for i in range(QB):
        t = in_ref[i].T
        t4 = t.reshape(2, 8, 8, BLK)
        out_ref[0, :, 2 * i, :, :] = t4[0]
        out_ref[0, :, 2 * i + 1, :, :] = t4[1]


def kernel(input_ids, table):
    ids4 = input_ids.T.reshape(SHI, 8, NW, BLK).transpose(0, 2, 1, 3)
    t2 = jax.lax.optimization_barrier(table.reshape(500000, 2 * EMBED))
    tlin = t2.reshape(1000000, EMBED)
    mesh = plsc.VectorSubcoreMesh(core_axis_name="c", subcore_axis_name="s")
    inter = pl.kernel(
        _sc_body,
        out_type=jax.ShapeDtypeStruct((NQ, BLK, BLK), jnp.float32),
        mesh=mesh,
        scratch_types=(
            [pltpu.VMEM((SHI, 8, BLK), jnp.int32),
             pltpu.VMEM((2 * H, BLK, EMBED), jnp.float32)]
            + [pltpu.SemaphoreType.DMA] * (4 * H)
        ),
        compiler_params=pltpu.CompilerParams(use_tc_tiling_on_sc=False,
                                             needs_layout_passes=False),
    )(ids4, tlin)

    out5 = pl.pallas_call(
        _tc_body,
        out_shape=jax.ShapeDtypeStruct((SEQ, 8, NW, 8, BLK), jnp.float32),
        grid=(NQ // QB,),
        in_specs=[pl.BlockSpec((QB, BLK, BLK), lambda t: (t, 0, 0))],
        out_specs=pl.BlockSpec((1, 8, 2 * QB, 8, BLK),
                               lambda t: (t, 0, 0, 0, 0)),
    )(inter)

    return out5.transpose(2, 4, 0, 1, 3).reshape(BATCH, SEQ, EMBED)
